# Initial kernel scaffold; baseline (speedup 1.0000x reference)
import jax
import jax.numpy as jnp
from jax.experimental import pallas as pl


def kernel(voxel, points, closest_points, quads):
    raise NotImplementedError("write your pallas kernel here")


# speedup vs baseline: 604.2004x; 604.2004x over previous
import jax
import jax.numpy as jnp
from jax import lax
from jax.experimental import pallas as pl
from jax.experimental.pallas import tpu as pltpu
from jax.experimental.pallas import tpu_sc as plsc

B = 8
N = 8192
Q = 16
G = 32
NW = 32
PPW = N * B // NW
CHUNK = 128
NCHUNK = PPW // CHUNK
SUB = CHUNK // 16


def _sc_body(coef_hbm, pts_hbm, cpx_hbm, cpy_hbm, cpz_hbm, vox_hbm, out_hbm,
             px, py, pz, coef_v, idx_v, gx, gy, gz, gv,
             rbx, rby, rbz, accv, sem):
    wid = lax.axis_index("s") * 2 + lax.axis_index("c")
    b = wid // 4
    p0 = (wid % 4) * PPW

    pltpu.sync_copy(pts_hbm.at[b, 0, pl.ds(p0, PPW)], px)
    pltpu.sync_copy(pts_hbm.at[b, 1, pl.ds(p0, PPW)], py)
    pltpu.sync_copy(pts_hbm.at[b, 2, pl.ds(p0, PPW)], pz)
    pltpu.sync_copy(coef_hbm.at[b], coef_v)

    base_idx = (b * (G * G * G)).astype(jnp.int32)

    def chunk_body(c, acc):
        qb = (wid % 4) * 4 + c // 4
        crow = coef_v[qb, :]
        m00 = crow[0]
        m01 = crow[1]
        m02 = crow[2]
        m10 = crow[3]
        m11 = crow[4]
        m12 = crow[5]
        m20 = crow[6]
        m21 = crow[7]
        m22 = crow[8]
        t0 = crow[9]
        t1 = crow[10]
        t2 = crow[11]

        def phase_a(i, _):
            off = pl.multiple_of(c * CHUNK + i * 16, 16)
            vx = px[pl.ds(off, 16)]
            vy = py[pl.ds(off, 16)]
            vz = pz[pl.ds(off, 16)]
            rx = m00 * vx + m01 * vy + m02 * vz + t0
            ry = m10 * vx + m11 * vy + m12 * vz + t1
            rz = m20 * vx + m21 * vy + m22 * vz + t2

            def vceil(t):
                ti = t.astype(jnp.int32)
                tf = ti.astype(jnp.float32)
                return ti + jnp.where(t > tf, 1, 0).astype(jnp.int32)

            ix = vceil((rx + 0.5) * G - 0.5)
            iy = vceil((ry + 0.5) * G - 0.5)
            iz = vceil((rz + 0.5) * G - 0.5)
            ind = ix * (G * G) + iy * G + iz
            ind = jnp.minimum(jnp.maximum(ind, 0), G * G * G - 1) + base_idx
            soff = pl.multiple_of(i * 16, 16)
            rbx[pl.ds(soff, 16)] = rx
            rby[pl.ds(soff, 16)] = ry
            rbz[pl.ds(soff, 16)] = rz
            idx_v[pl.ds(soff, 16)] = ind
            return 0

        lax.fori_loop(0, SUB, phase_a, 0)

        d1 = pltpu.async_copy(cpx_hbm.at[idx_v], gx, sem)
        d2 = pltpu.async_copy(cpy_hbm.at[idx_v], gy, sem)
        d3 = pltpu.async_copy(cpz_hbm.at[idx_v], gz, sem)
        d4 = pltpu.async_copy(vox_hbm.at[idx_v], gv, sem)
        d1.wait()
        d2.wait()
        d3.wait()
        d4.wait()

        def phase_c(i, acc):
            soff = pl.multiple_of(i * 16, 16)
            m = 1.0 - gv[pl.ds(soff, 16)]
            dx = (rbx[pl.ds(soff, 16)] - gx[pl.ds(soff, 16)]) * m
            dy = (rby[pl.ds(soff, 16)] - gy[pl.ds(soff, 16)]) * m
            dz = (rbz[pl.ds(soff, 16)] - gz[pl.ds(soff, 16)]) * m
            return acc + (dx * dx + dy * dy + dz * dz)

        return lax.fori_loop(0, SUB, phase_c, acc)

    acc = lax.fori_loop(0, NCHUNK, chunk_body, jnp.zeros((16,), jnp.float32))
    accv[...] = acc
    pltpu.sync_copy(accv, out_hbm.at[wid])


def _tc_reduce_body(x_ref, o_ref):
    o_ref[0, 0] = jnp.sum(x_ref[...]) * (1.0 / (B * N))


def kernel(voxel, points, closest_points, quads):
    mid = jnp.mean(points, axis=1)
    qs = quads[..., 1:]
    qs = qs / jnp.linalg.norm(qs, ord=2, axis=2, keepdims=True)
    qs = jnp.concatenate([jnp.ones((B, Q, 1), jnp.float32), qs], axis=-1)
    qs = 0.707 * qs
    w, x, y, z = qs[..., 0], qs[..., 1], qs[..., 2], qs[..., 3]
    M = jnp.stack([
        w * w + x * x - y * y - z * z, 2 * (x * y - w * z), 2 * (x * z + w * y),
        2 * (x * y + w * z), w * w - x * x + y * y - z * z, 2 * (y * z - w * x),
        2 * (x * z - w * y), 2 * (y * z + w * x), w * w - x * x - y * y + z * z,
    ], axis=-1).reshape(B, Q, 3, 3)
    t = -jnp.einsum("bqij,bj->bqi", M, mid)
    coef = jnp.concatenate(
        [M.reshape(B, Q, 9), t, jnp.zeros((B, Q, 4), jnp.float32)], axis=-1)

    pts_t = points.transpose(0, 2, 1)
    cpx = closest_points[..., 0].reshape(B * G * G * G)
    cpy = closest_points[..., 1].reshape(B * G * G * G)
    cpz = closest_points[..., 2].reshape(B * G * G * G)
    vox = voxel.reshape(B * G * G * G)

    mesh = plsc.VectorSubcoreMesh(core_axis_name="c", subcore_axis_name="s")
    partials = pl.kernel(
        _sc_body,
        out_type=jax.ShapeDtypeStruct((NW, 16), jnp.float32),
        mesh=mesh,
        compiler_params=pltpu.CompilerParams(
            use_tc_tiling_on_sc=False, needs_layout_passes=False),
        scratch_types=[
            pltpu.VMEM((PPW,), jnp.float32),
            pltpu.VMEM((PPW,), jnp.float32),
            pltpu.VMEM((PPW,), jnp.float32),
            pltpu.VMEM((Q, 16), jnp.float32),
            pltpu.VMEM((CHUNK,), jnp.int32),
            pltpu.VMEM((CHUNK,), jnp.float32),
            pltpu.VMEM((CHUNK,), jnp.float32),
            pltpu.VMEM((CHUNK,), jnp.float32),
            pltpu.VMEM((CHUNK,), jnp.float32),
            pltpu.VMEM((CHUNK,), jnp.float32),
            pltpu.VMEM((CHUNK,), jnp.float32),
            pltpu.VMEM((CHUNK,), jnp.float32),
            pltpu.VMEM((16,), jnp.float32),
            pltpu.SemaphoreType.DMA,
        ],
    )(coef, pts_t, cpx, cpy, cpz, vox)

    total = pl.pallas_call(
        _tc_reduce_body,
        out_shape=jax.ShapeDtypeStruct((1, 1), jnp.float32),
        out_specs=pl.BlockSpec(memory_space=pltpu.SMEM),
    )(partials)
    return total[0, 0]


# speedup vs baseline: 687.2421x; 1.1374x over previous
import jax
import jax.numpy as jnp
from jax import lax
from jax.experimental import pallas as pl
from jax.experimental.pallas import tpu as pltpu
from jax.experimental.pallas import tpu_sc as plsc

B = 8
N = 8192
Q = 16
G = 32
NW = 32
PPW = N * B // NW
CHUNK = 128
NCHUNK = PPW // CHUNK
SUB = CHUNK // 16


def _sc_body(coef_hbm, pts_hbm, cpx_hbm, cpy_hbm, cpz_hbm, vox_hbm, out_hbm,
             px, py, pz, coef_v, idx_v, gx, gy, gz, gv,
             rbx, rby, rbz, accv, sem):
    wid = lax.axis_index("s") * 2 + lax.axis_index("c")
    b = wid // 4
    p0 = (wid % 4) * PPW

    pltpu.sync_copy(pts_hbm.at[b, 0, pl.ds(p0, PPW)], px)
    pltpu.sync_copy(pts_hbm.at[b, 1, pl.ds(p0, PPW)], py)
    pltpu.sync_copy(pts_hbm.at[b, 2, pl.ds(p0, PPW)], pz)
    pltpu.sync_copy(coef_hbm.at[b], coef_v)

    base_idx = (b * (G * G * G)).astype(jnp.int32)

    copies = []
    for c in range(NCHUNK):
        qb = (wid % 4) * 4 + c // 4
        crow = coef_v[qb, :]
        m00 = crow[0]
        m01 = crow[1]
        m02 = crow[2]
        m10 = crow[3]
        m11 = crow[4]
        m12 = crow[5]
        m20 = crow[6]
        m21 = crow[7]
        m22 = crow[8]
        t0 = crow[9]
        t1 = crow[10]
        t2 = crow[11]

        def phase_a(i, _, c=c, m00=m00, m01=m01, m02=m02, m10=m10, m11=m11,
                    m12=m12, m20=m20, m21=m21, m22=m22, t0=t0, t1=t1, t2=t2):
            off = pl.multiple_of(c * CHUNK + i * 16, 16)
            vx = px[pl.ds(off, 16)]
            vy = py[pl.ds(off, 16)]
            vz = pz[pl.ds(off, 16)]
            rx = m00 * vx + m01 * vy + m02 * vz + t0
            ry = m10 * vx + m11 * vy + m12 * vz + t1
            rz = m20 * vx + m21 * vy + m22 * vz + t2

            def vceil(t):
                ti = t.astype(jnp.int32)
                tf = ti.astype(jnp.float32)
                return ti + jnp.where(t > tf, 1, 0).astype(jnp.int32)

            ix = vceil((rx + 0.5) * G - 0.5)
            iy = vceil((ry + 0.5) * G - 0.5)
            iz = vceil((rz + 0.5) * G - 0.5)
            ind = ix * (G * G) + iy * G + iz
            ind = jnp.minimum(jnp.maximum(ind, 0), G * G * G - 1) + base_idx
            soff = pl.multiple_of(i * 16, 16)
            rbx[c, pl.ds(soff, 16)] = rx
            rby[c, pl.ds(soff, 16)] = ry
            rbz[c, pl.ds(soff, 16)] = rz
            idx_v[c, pl.ds(soff, 16)] = ind
            return 0

        lax.fori_loop(0, SUB, phase_a, 0)

        copies.append((
            pltpu.async_copy(cpx_hbm.at[idx_v.at[c]], gx.at[c], sem),
            pltpu.async_copy(cpy_hbm.at[idx_v.at[c]], gy.at[c], sem),
            pltpu.async_copy(cpz_hbm.at[idx_v.at[c]], gz.at[c], sem),
            pltpu.async_copy(vox_hbm.at[idx_v.at[c]], gv.at[c], sem),
        ))

    acc = jnp.zeros((16,), jnp.float32)
    for c in range(NCHUNK):
        for d in copies[c]:
            d.wait()

        def phase_c(i, acc, c=c):
            soff = pl.multiple_of(i * 16, 16)
            m = 1.0 - gv[c, pl.ds(soff, 16)]
            dx = (rbx[c, pl.ds(soff, 16)] - gx[c, pl.ds(soff, 16)]) * m
            dy = (rby[c, pl.ds(soff, 16)] - gy[c, pl.ds(soff, 16)]) * m
            dz = (rbz[c, pl.ds(soff, 16)] - gz[c, pl.ds(soff, 16)]) * m
            return acc + (dx * dx + dy * dy + dz * dz)

        acc = lax.fori_loop(0, SUB, phase_c, acc)

    accv[...] = acc
    pltpu.sync_copy(accv, out_hbm.at[wid])


def _tc_reduce_body(x_ref, o_ref):
    o_ref[0, 0] = jnp.sum(x_ref[...]) * (1.0 / (B * N))


def kernel(voxel, points, closest_points, quads):
    mid = jnp.mean(points, axis=1)
    qs = quads[..., 1:]
    qs = qs / jnp.linalg.norm(qs, ord=2, axis=2, keepdims=True)
    qs = jnp.concatenate([jnp.ones((B, Q, 1), jnp.float32), qs], axis=-1)
    qs = 0.707 * qs
    w, x, y, z = qs[..., 0], qs[..., 1], qs[..., 2], qs[..., 3]
    M = jnp.stack([
        w * w + x * x - y * y - z * z, 2 * (x * y - w * z), 2 * (x * z + w * y),
        2 * (x * y + w * z), w * w - x * x + y * y - z * z, 2 * (y * z - w * x),
        2 * (x * z - w * y), 2 * (y * z + w * x), w * w - x * x - y * y + z * z,
    ], axis=-1).reshape(B, Q, 3, 3)
    t = -jnp.einsum("bqij,bj->bqi", M, mid)
    coef = jnp.concatenate(
        [M.reshape(B, Q, 9), t, jnp.zeros((B, Q, 4), jnp.float32)], axis=-1)

    pts_t = points.transpose(0, 2, 1)
    cpx = closest_points[..., 0].reshape(B * G * G * G)
    cpy = closest_points[..., 1].reshape(B * G * G * G)
    cpz = closest_points[..., 2].reshape(B * G * G * G)
    vox = voxel.reshape(B * G * G * G)

    mesh = plsc.VectorSubcoreMesh(core_axis_name="c", subcore_axis_name="s")
    partials = pl.kernel(
        _sc_body,
        out_type=jax.ShapeDtypeStruct((NW, 16), jnp.float32),
        mesh=mesh,
        compiler_params=pltpu.CompilerParams(
            use_tc_tiling_on_sc=False, needs_layout_passes=False),
        scratch_types=[
            pltpu.VMEM((PPW,), jnp.float32),
            pltpu.VMEM((PPW,), jnp.float32),
            pltpu.VMEM((PPW,), jnp.float32),
            pltpu.VMEM((Q, 16), jnp.float32),
            pltpu.VMEM((NCHUNK, CHUNK), jnp.int32),
            pltpu.VMEM((NCHUNK, CHUNK), jnp.float32),
            pltpu.VMEM((NCHUNK, CHUNK), jnp.float32),
            pltpu.VMEM((NCHUNK, CHUNK), jnp.float32),
            pltpu.VMEM((NCHUNK, CHUNK), jnp.float32),
            pltpu.VMEM((NCHUNK, CHUNK), jnp.float32),
            pltpu.VMEM((NCHUNK, CHUNK), jnp.float32),
            pltpu.VMEM((NCHUNK, CHUNK), jnp.float32),
            pltpu.VMEM((16,), jnp.float32),
            pltpu.SemaphoreType.DMA,
        ],
    )(coef, pts_t, cpx, cpy, cpz, vox)

    total = pl.pallas_call(
        _tc_reduce_body,
        out_shape=jax.ShapeDtypeStruct((1, 1), jnp.float32),
        out_specs=pl.BlockSpec(memory_space=pltpu.SMEM),
    )(partials)
    return total[0, 0]
